# Initial kernel scaffold; baseline (speedup 1.0000x reference)
import jax
import jax.numpy as jnp
from jax.experimental import pallas as pl


def kernel(embedding, edge_nodes, Wp, bp, Wq, bq, W1, b1, W2, b2):
    raise NotImplementedError("write your pallas kernel here")


# speedup vs baseline: 3.8145x; 3.8145x over previous
import functools
import math

import jax
import jax.numpy as jnp
from jax import lax
from jax.experimental import pallas as pl
from jax.experimental.pallas import tpu as pltpu
from jax.experimental.pallas import tpu_sc as plsc

N_NODES = 50000
FEATDIM = 128
N_EDGES = 100000
EDGE_ORDER = 4
RANK = 50
OUTDIM = 64
HIDDEN = 128

NC = 2
NS = 16
NW = NC * NS

N_PAD = 50176
E_PAD = 102400
M_PAD = E_PAD * EDGE_ORDER

ROWS_PER_TILE = N_PAD // NW
SROWS = N_PAD // NS
M_PER_W = M_PAD // NW
GCHUNK = 512
G_ITERS = M_PER_W // GCHUNK
M_PER_T = M_PAD // NS
N_PAD4 = N_PAD // 4
SROWS4 = N_PAD4 // NS
SCHUNK = 128
S_ITERS = M_PER_T // SCHUNK
ZROWS = 56
Z_ITERS = SROWS4 // ZROWS

def _wid():
    return lax.axis_index("s") * NC + lax.axis_index("c")


def _deg_body(idx_hbm, ones_hbm, zeros_hbm, degA_hbm, degB_hbm,
              idx_v, ones_v, deg_sh):
    c = lax.axis_index("c")
    wid = _wid()
    rbase = lax.axis_index("s") * SROWS
    stage = ones_v.at[pl.ds(0, SROWS)]
    pltpu.sync_copy(zeros_hbm.at[pl.ds(rbase, SROWS)], stage)
    pltpu.sync_copy(stage, deg_sh.at[pl.ds(rbase, SROWS)])
    plsc.subcore_barrier()
    pltpu.sync_copy(idx_hbm.at[pl.ds(wid * M_PER_W, M_PER_W)], idx_v)
    pltpu.sync_copy(ones_hbm, ones_v)
    pltpu.sync_copy(ones_v, deg_sh.at[idx_v], add=True)
    plsc.subcore_barrier()
    pltpu.sync_copy(deg_sh.at[pl.ds(rbase, SROWS)], stage)

    @pl.when(c == 0)
    def _():
        pltpu.sync_copy(stage, degA_hbm.at[pl.ds(rbase, SROWS)])

    @pl.when(c == 1)
    def _():
        pltpu.sync_copy(stage, degB_hbm.at[pl.ds(rbase, SROWS)])


def _gather_body(idx_hbm, c_hbm, g_hbm, idx_v, r_v, sem):
    base = _wid() * M_PER_W

    def body(i, _):
        off = base + i * GCHUNK
        pltpu.sync_copy(idx_hbm.at[pl.ds(off, GCHUNK)], idx_v)
        pltpu.async_copy(c_hbm.at[idx_v], r_v, sem).wait()
        pltpu.sync_copy(r_v, g_hbm.at[pl.ds(off, GCHUNK)])
        return 0

    lax.fori_loop(0, G_ITERS, body, 0)


def _scatter_body(idxq_hbm, vL_hbm, vR_hbm, zeros_hbm, sumL_hbm, sumR_hbm,
                  idx_v, val_v, stage_v, acc_sh):
    c = lax.axis_index("c")
    s = lax.axis_index("s")
    rbase = s * SROWS4

    def zinit(i, _):
        pltpu.sync_copy(zeros_hbm, stage_v)
        pltpu.sync_copy(stage_v, acc_sh.at[pl.ds(rbase + i * ZROWS, ZROWS)])
        return 0

    lax.fori_loop(0, Z_ITERS, zinit, 0)
    plsc.subcore_barrier()

    def scatter_all(v_hbm):
        def body(i, _):
            off = s * M_PER_T + i * SCHUNK
            pltpu.sync_copy(idxq_hbm.at[pl.ds(off, SCHUNK)], idx_v)
            pltpu.sync_copy(v_hbm.at[pl.ds(off, SCHUNK)], val_v)
            pltpu.sync_copy(val_v, acc_sh.at[idx_v], add=True)
            return 0
        lax.fori_loop(0, S_ITERS, body, 0)

    @pl.when(c == 0)
    def _():
        scatter_all(vL_hbm)

    @pl.when(c == 1)
    def _():
        scatter_all(vR_hbm)

    plsc.subcore_barrier()

    def dump(out_hbm):
        def body(i, _):
            r0 = rbase + i * ZROWS
            pltpu.sync_copy(acc_sh.at[pl.ds(r0, ZROWS)], stage_v)
            pltpu.sync_copy(stage_v, out_hbm.at[pl.ds(r0, ZROWS)])
            return 0
        lax.fori_loop(0, Z_ITERS, body, 0)

    @pl.when(c == 0)
    def _():
        dump(sumL_hbm)

    @pl.when(c == 1)
    def _():
        dump(sumR_hbm)


@functools.lru_cache(maxsize=1)
def _sc_kernels():
    mesh = plsc.VectorSubcoreMesh(
        core_axis_name="c", subcore_axis_name="s", num_cores=NC, num_subcores=NS
    )
    deg = pl.kernel(
        _deg_body,
        out_type=(
            jax.ShapeDtypeStruct((N_PAD,), jnp.float32),
            jax.ShapeDtypeStruct((N_PAD,), jnp.float32),
        ),
        mesh=mesh,
        scratch_types=[
            pltpu.VMEM((M_PER_W,), jnp.int32),
            pltpu.VMEM((M_PER_W,), jnp.float32),
            pltpu.VMEM_SHARED((N_PAD,), jnp.float32),
        ],
    )
    gather = pl.kernel(
        _gather_body,
        out_type=jax.ShapeDtypeStruct((M_PAD, 128), jnp.float32),
        mesh=mesh,
        scratch_types=[
            pltpu.VMEM((GCHUNK,), jnp.int32),
            pltpu.VMEM((GCHUNK, 128), jnp.float32),
            pltpu.SemaphoreType.DMA,
        ],
    )
    scatter = pl.kernel(
        _scatter_body,
        out_type=(
            jax.ShapeDtypeStruct((N_PAD4, 128), jnp.float32),
            jax.ShapeDtypeStruct((N_PAD4, 128), jnp.float32),
        ),
        mesh=mesh,
        scratch_types=[
            pltpu.VMEM((SCHUNK,), jnp.int32),
            pltpu.VMEM((SCHUNK, 128), jnp.float32),
            pltpu.VMEM((ZROWS, 128), jnp.float32),
            pltpu.VMEM_SHARED((N_PAD4, 128), jnp.float32),
        ],
    )
    return deg, gather, scatter


def _tables_body(emb_ref, dA_ref, dB_ref, wp_ref, bp_ref, w1_ref, b1_ref,
                 w2_ref, b2_ref, embL_ref, embR_ref, c_ref):
    emb = emb_ref[...]
    deg = jnp.maximum(dA_ref[...] + dB_ref[...], 1.0)
    t = jnp.sqrt(jnp.sqrt(deg))
    lanes = lax.broadcasted_iota(jnp.int32, (emb.shape[0], 64), 1)
    first = jnp.dot(emb, wp_ref[...], preferred_element_type=jnp.float32)
    first = first + bp_ref[...]
    first = jnp.where(lanes == 63, t, first)
    h = jnp.maximum(
        jnp.dot(emb, w1_ref[...], preferred_element_type=jnp.float32)
        + b1_ref[...], 0.0)
    c2 = jnp.dot(h, w2_ref[...],
                 preferred_element_type=jnp.float32) + b2_ref[...]
    c_ref[...] = (
        jnp.dot(first, embL_ref[...], preferred_element_type=jnp.float32)
        + jnp.dot(c2, embR_ref[...], preferred_element_type=jnp.float32))


def _build_tables(emb_pad, dA, dB, wp_pad, bp_pad, w1, b1r, w2, b2r,
                  embL, embR):
    bn = 512
    grid = (N_PAD // bn,)
    return pl.pallas_call(
        _tables_body,
        grid=grid,
        in_specs=[
            pl.BlockSpec((bn, FEATDIM), lambda i: (i, 0)),
            pl.BlockSpec((bn, 1), lambda i: (i, 0)),
            pl.BlockSpec((bn, 1), lambda i: (i, 0)),
            pl.BlockSpec((FEATDIM, 64), lambda i: (0, 0)),
            pl.BlockSpec((1, 64), lambda i: (0, 0)),
            pl.BlockSpec((FEATDIM, HIDDEN), lambda i: (0, 0)),
            pl.BlockSpec((1, HIDDEN), lambda i: (0, 0)),
            pl.BlockSpec((HIDDEN, 64), lambda i: (0, 0)),
            pl.BlockSpec((1, 64), lambda i: (0, 0)),
            pl.BlockSpec((64, 128), lambda i: (0, 0)),
            pl.BlockSpec((64, 128), lambda i: (0, 0)),
        ],
        out_specs=pl.BlockSpec((bn, 128), lambda i: (i, 0)),
        out_shape=jax.ShapeDtypeStruct((N_PAD, 128), jnp.float32),
    )(emb_pad, dA, dB, wp_pad, bp_pad, w1, b1r, w2, b2r, embL, embR)


def _combine_body_split(g_ref, en_ref, wq_ref, bq_ref, oL_ref, oR_ref):
    g = g_ref[...]
    en = en_ref[...]
    be = g.shape[1]
    g1 = g[:, :, :64]
    g2 = g[:, :, 64:]
    e0, e1, e2, e3 = g1[0], g1[1], g1[2], g1[3]
    s01 = e0[:, 63:64] * e1[:, 63:64]
    s23 = e2[:, 63:64] * e3[:, 63:64]
    cs = (1.0 / 6.0) * (s01 * s23)
    a01 = e0 * e1
    a23 = e2 * e3
    wq = wq_ref[...]
    bq = bq_ref[...]
    edge2 = jnp.maximum(g2[0] + g2[1] + g2[2] + g2[3], 0.0) + bq
    loos = (e1 * a23, e0 * a23, a01 * e3, a01 * e2)
    blocks = lax.broadcasted_iota(jnp.int32, (be, 128), 1) // 32
    for j in range(EDGE_ORDER):
        pre = jnp.tanh(cs * loos[j])
        out = jnp.dot(pre, wq, preferred_element_type=jnp.float32) + edge2
        mask = blocks == (en[j] % 4).reshape(be, 1)
        outL = jnp.concatenate([out[:, :32]] * 4, axis=1)
        outR = jnp.concatenate([out[:, 32:]] * 4, axis=1)
        oL_ref[j, :, :] = jnp.where(mask, outL, 0.0)
        oR_ref[j, :, :] = jnp.where(mask, outR, 0.0)


def _combine(g, en_m, wq_pad, bq_pad):
    be = 512
    grid = (E_PAD // be,)
    return pl.pallas_call(
        _combine_body_split,
        grid=grid,
        in_specs=[
            pl.BlockSpec((EDGE_ORDER, be, 128), lambda i: (0, i, 0)),
            pl.BlockSpec((EDGE_ORDER, be), lambda i: (0, i)),
            pl.BlockSpec((64, 64), lambda i: (0, 0)),
            pl.BlockSpec((1, 64), lambda i: (0, 0)),
        ],
        out_specs=[
            pl.BlockSpec((EDGE_ORDER, be, 128), lambda i: (0, i, 0)),
            pl.BlockSpec((EDGE_ORDER, be, 128), lambda i: (0, i, 0)),
        ],
        out_shape=[
            jax.ShapeDtypeStruct((EDGE_ORDER, E_PAD, 128), jnp.float32),
            jax.ShapeDtypeStruct((EDGE_ORDER, E_PAD, 128), jnp.float32),
        ],
    )(g, en_m, wq_pad, bq_pad)


def _final_body(sL_ref, sR_ref, dA_ref, dB_ref, eL_ref, eR_ref, o_ref):
    deg = jnp.maximum(dA_ref[...] + dB_ref[...], 1.0)
    full = (jnp.dot(sL_ref[...], eL_ref[...], preferred_element_type=jnp.float32)
            + jnp.dot(sR_ref[...], eR_ref[...], preferred_element_type=jnp.float32))
    o_ref[...] = jnp.maximum(full / deg, 0.0)


def _finalize(sumL, sumR, dA, dB, eyeL, eyeR):
    bn = 1000
    grid = (N_NODES // bn,)
    return pl.pallas_call(
        _final_body,
        grid=grid,
        in_specs=[
            pl.BlockSpec((bn, 32), lambda i: (i, 0)),
            pl.BlockSpec((bn, 32), lambda i: (i, 0)),
            pl.BlockSpec((bn, 1), lambda i: (i, 0)),
            pl.BlockSpec((bn, 1), lambda i: (i, 0)),
            pl.BlockSpec((32, 64), lambda i: (0, 0)),
            pl.BlockSpec((32, 64), lambda i: (0, 0)),
        ],
        out_specs=pl.BlockSpec((bn, 64), lambda i: (i, 0)),
        out_shape=jax.ShapeDtypeStruct((N_NODES, 64), jnp.float32),
    )(sumL, sumR, dA, dB, eyeL, eyeR)


def kernel(embedding, edge_nodes, Wp, bp, Wq, bq, W1, b1, W2, b2):
    f32 = jnp.float32
    en = edge_nodes.astype(jnp.int32)
    en_pad = jnp.concatenate(
        [en, jnp.full((E_PAD - N_EDGES, EDGE_ORDER), N_NODES, jnp.int32)], axis=0)
    idx_flat = en_pad.T.reshape(-1)

    idx_q = idx_flat // 4
    emb_pad = jnp.zeros((N_PAD, FEATDIM), f32).at[:N_NODES].set(embedding)
    zeros_vec = jnp.zeros((N_PAD,), f32)
    zeros_mat = jnp.zeros((ZROWS, 128), f32)
    ones_vec = jnp.ones((M_PER_W,), f32)
    wp_pad = jnp.zeros((FEATDIM, 64), f32).at[:, :RANK].set(Wp)
    bp_pad = jnp.zeros((1, 64), f32).at[0, :RANK].set(bp)
    wq_pad = jnp.zeros((64, 64), f32).at[:RANK, :].set(Wq)
    bq_pad = bq.reshape(1, OUTDIM).astype(f32)
    b1r = b1.reshape(1, HIDDEN).astype(f32)
    b2r = b2.reshape(1, OUTDIM).astype(f32)
    eye = jnp.eye(32, dtype=f32)
    eyeL = jnp.zeros((32, 64), f32).at[:, :32].set(eye)
    eyeR = jnp.zeros((32, 64), f32).at[:, 32:].set(eye)
    eye64 = jnp.eye(64, dtype=f32)
    embL = jnp.zeros((64, 128), f32).at[:, :64].set(eye64)
    embR = jnp.zeros((64, 128), f32).at[:, 64:].set(eye64)

    deg_k, gather_k, scatter_k = _sc_kernels()
    degA, degB = deg_k(idx_flat, ones_vec, zeros_vec)
    dA = degA.reshape(N_PAD, 1)
    dB = degB.reshape(N_PAD, 1)

    c_table = _build_tables(emb_pad, dA, dB, wp_pad, bp_pad, W1, b1r, W2, b2r,
                            embL, embR)

    g = gather_k(idx_flat, c_table)

    cL, cR = _combine(g.reshape(EDGE_ORDER, E_PAD, 128),
                      en_pad.T.reshape(EDGE_ORDER, E_PAD), wq_pad, bq_pad)

    sumL, sumR = scatter_k(idx_q, cL.reshape(M_PAD, 128),
                           cR.reshape(M_PAD, 128), zeros_mat)

    return _finalize(sumL.reshape(N_PAD, 32), sumR.reshape(N_PAD, 32),
                     dA, dB, eyeL, eyeR)


# speedup vs baseline: 4.0159x; 1.0528x over previous
import functools
import math

import jax
import jax.numpy as jnp
from jax import lax
from jax.experimental import pallas as pl
from jax.experimental.pallas import tpu as pltpu
from jax.experimental.pallas import tpu_sc as plsc

N_NODES = 50000
FEATDIM = 128
N_EDGES = 100000
EDGE_ORDER = 4
RANK = 50
OUTDIM = 64
HIDDEN = 128

NC = 2
NS = 16
NW = NC * NS

N_PAD = 50176
E_PAD = 102400
M_PAD = E_PAD * EDGE_ORDER

ROWS_PER_TILE = N_PAD // NW
SROWS = N_PAD // NS
M_PER_W = M_PAD // NW
GCHUNK = 512
G_ITERS = M_PER_W // GCHUNK
M_PER_T = M_PAD // NS
N_PAD4 = N_PAD // 4
SROWS4 = N_PAD4 // NS
SCHUNK = 128
S_ITERS = M_PER_T // SCHUNK
ZROWS = 56
Z_ITERS = SROWS4 // ZROWS

def _wid():
    return lax.axis_index("s") * NC + lax.axis_index("c")


def _deg_body(idx_hbm, ones_hbm, zeros_hbm, degA_hbm, degB_hbm,
              idx_v, ones_v, deg_sh):
    c = lax.axis_index("c")
    wid = _wid()
    rbase = lax.axis_index("s") * SROWS
    stage = ones_v.at[pl.ds(0, SROWS)]
    pltpu.sync_copy(zeros_hbm.at[pl.ds(rbase, SROWS)], stage)
    pltpu.sync_copy(stage, deg_sh.at[pl.ds(rbase, SROWS)])
    plsc.subcore_barrier()
    pltpu.sync_copy(idx_hbm.at[pl.ds(wid * M_PER_W, M_PER_W)], idx_v)
    pltpu.sync_copy(ones_hbm, ones_v)
    pltpu.sync_copy(ones_v, deg_sh.at[idx_v], add=True)
    plsc.subcore_barrier()
    pltpu.sync_copy(deg_sh.at[pl.ds(rbase, SROWS)], stage)

    @pl.when(c == 0)
    def _():
        pltpu.sync_copy(stage, degA_hbm.at[pl.ds(rbase, SROWS)])

    @pl.when(c == 1)
    def _():
        pltpu.sync_copy(stage, degB_hbm.at[pl.ds(rbase, SROWS)])


def _gather_body(idx_hbm, c_hbm, g_hbm, idx_v, r_v, sem):
    base = _wid() * M_PER_W

    def body(i, _):
        off = base + i * GCHUNK
        pltpu.sync_copy(idx_hbm.at[pl.ds(off, GCHUNK)], idx_v)
        pltpu.async_copy(c_hbm.at[idx_v], r_v, sem).wait()
        pltpu.sync_copy(r_v, g_hbm.at[pl.ds(off, GCHUNK)])
        return 0

    lax.fori_loop(0, G_ITERS, body, 0)


def _scatter_body(idxE_hbm, idxO_hbm, v_hbm, zeros_hbm, sumE_hbm, sumO_hbm,
                  idx_v, val_v, stage_v, acc_sh):
    c = lax.axis_index("c")
    s = lax.axis_index("s")
    rbase = s * SROWS4

    def zinit(i, _):
        pltpu.sync_copy(zeros_hbm, stage_v)
        pltpu.sync_copy(stage_v, acc_sh.at[pl.ds(rbase + i * ZROWS, ZROWS)])
        return 0

    lax.fori_loop(0, Z_ITERS, zinit, 0)
    plsc.subcore_barrier()

    def scatter_all(idx_hbm):
        def body(i, _):
            off = s * M_PER_T + i * SCHUNK
            pltpu.sync_copy(idx_hbm.at[pl.ds(off, SCHUNK)], idx_v)
            pltpu.sync_copy(v_hbm.at[pl.ds(off, SCHUNK)], val_v)
            pltpu.sync_copy(val_v, acc_sh.at[idx_v], add=True)
            return 0
        lax.fori_loop(0, S_ITERS, body, 0)

    @pl.when(c == 0)
    def _():
        scatter_all(idxE_hbm)

    @pl.when(c == 1)
    def _():
        scatter_all(idxO_hbm)

    plsc.subcore_barrier()

    def dump(out_hbm):
        def body(i, _):
            r0 = rbase + i * ZROWS
            pltpu.sync_copy(acc_sh.at[pl.ds(r0, ZROWS)], stage_v)
            pltpu.sync_copy(stage_v, out_hbm.at[pl.ds(r0, ZROWS)])
            return 0
        lax.fori_loop(0, Z_ITERS, body, 0)

    @pl.when(c == 0)
    def _():
        dump(sumE_hbm)

    @pl.when(c == 1)
    def _():
        dump(sumO_hbm)


@functools.lru_cache(maxsize=1)
def _sc_kernels():
    mesh = plsc.VectorSubcoreMesh(
        core_axis_name="c", subcore_axis_name="s", num_cores=NC, num_subcores=NS
    )
    deg = pl.kernel(
        _deg_body,
        out_type=(
            jax.ShapeDtypeStruct((N_PAD,), jnp.float32),
            jax.ShapeDtypeStruct((N_PAD,), jnp.float32),
        ),
        mesh=mesh,
        scratch_types=[
            pltpu.VMEM((M_PER_W,), jnp.int32),
            pltpu.VMEM((M_PER_W,), jnp.float32),
            pltpu.VMEM_SHARED((N_PAD,), jnp.float32),
        ],
    )
    gather = pl.kernel(
        _gather_body,
        out_type=jax.ShapeDtypeStruct((M_PAD, 128), jnp.float32),
        mesh=mesh,
        scratch_types=[
            pltpu.VMEM((GCHUNK,), jnp.int32),
            pltpu.VMEM((GCHUNK, 128), jnp.float32),
            pltpu.SemaphoreType.DMA,
        ],
    )
    scatter = pl.kernel(
        _scatter_body,
        out_type=(
            jax.ShapeDtypeStruct((N_PAD4, 128), jnp.float32),
            jax.ShapeDtypeStruct((N_PAD4, 128), jnp.float32),
        ),
        mesh=mesh,
        scratch_types=[
            pltpu.VMEM((SCHUNK,), jnp.int32),
            pltpu.VMEM((SCHUNK, 128), jnp.float32),
            pltpu.VMEM((ZROWS, 128), jnp.float32),
            pltpu.VMEM_SHARED((N_PAD4, 128), jnp.float32),
        ],
    )
    return deg, gather, scatter


def _tables_body(emb_ref, dA_ref, dB_ref, wp_ref, bp_ref, w1_ref, b1_ref,
                 w2_ref, b2_ref, embL_ref, embR_ref, c_ref):
    emb = emb_ref[...]
    deg = jnp.maximum(dA_ref[...] + dB_ref[...], 1.0)
    t = jnp.sqrt(jnp.sqrt(deg))
    lanes = lax.broadcasted_iota(jnp.int32, (emb.shape[0], 64), 1)
    first = jnp.dot(emb, wp_ref[...], preferred_element_type=jnp.float32)
    first = first + bp_ref[...]
    first = jnp.where(lanes == 63, t, first)
    h = jnp.maximum(
        jnp.dot(emb, w1_ref[...], preferred_element_type=jnp.float32)
        + b1_ref[...], 0.0)
    c2 = jnp.dot(h, w2_ref[...],
                 preferred_element_type=jnp.float32) + b2_ref[...]
    c_ref[...] = (
        jnp.dot(first, embL_ref[...], preferred_element_type=jnp.float32)
        + jnp.dot(c2, embR_ref[...], preferred_element_type=jnp.float32))


def _build_tables(emb_pad, dA, dB, wp_pad, bp_pad, w1, b1r, w2, b2r,
                  embL, embR):
    bn = 512
    grid = (N_PAD // bn,)
    return pl.pallas_call(
        _tables_body,
        grid=grid,
        in_specs=[
            pl.BlockSpec((bn, FEATDIM), lambda i: (i, 0)),
            pl.BlockSpec((bn, 1), lambda i: (i, 0)),
            pl.BlockSpec((bn, 1), lambda i: (i, 0)),
            pl.BlockSpec((FEATDIM, 64), lambda i: (0, 0)),
            pl.BlockSpec((1, 64), lambda i: (0, 0)),
            pl.BlockSpec((FEATDIM, HIDDEN), lambda i: (0, 0)),
            pl.BlockSpec((1, HIDDEN), lambda i: (0, 0)),
            pl.BlockSpec((HIDDEN, 64), lambda i: (0, 0)),
            pl.BlockSpec((1, 64), lambda i: (0, 0)),
            pl.BlockSpec((64, 128), lambda i: (0, 0)),
            pl.BlockSpec((64, 128), lambda i: (0, 0)),
        ],
        out_specs=pl.BlockSpec((bn, 128), lambda i: (i, 0)),
        out_shape=jax.ShapeDtypeStruct((N_PAD, 128), jnp.float32),
    )(emb_pad, dA, dB, wp_pad, bp_pad, w1, b1r, w2, b2r, embL, embR)


def _combine_body_split(g_ref, en_ref, wq_ref, bq_ref, o_ref):
    g = g_ref[...]
    en = en_ref[...]
    be = g.shape[1]
    g1 = g[:, :, :64]
    g2 = g[:, :, 64:]
    e0, e1, e2, e3 = g1[0], g1[1], g1[2], g1[3]
    s01 = e0[:, 63:64] * e1[:, 63:64]
    s23 = e2[:, 63:64] * e3[:, 63:64]
    cs = (1.0 / 6.0) * (s01 * s23)
    a01 = e0 * e1
    a23 = e2 * e3
    wq = wq_ref[...]
    bq = bq_ref[...]
    edge2 = jnp.maximum(g2[0] + g2[1] + g2[2] + g2[3], 0.0) + bq
    loos = (e1 * a23, e0 * a23, a01 * e3, a01 * e2)
    blocks = lax.broadcasted_iota(jnp.int32, (be, 128), 1) // 64
    for j in range(EDGE_ORDER):
        pre = jnp.tanh(cs * loos[j])
        out = jnp.dot(pre, wq, preferred_element_type=jnp.float32) + edge2
        mask = blocks == (en[j] % 2).reshape(be, 1)
        o_ref[j, :, :] = jnp.where(mask, jnp.concatenate([out, out], axis=1),
                                   0.0)


def _combine(g, en_m, wq_pad, bq_pad):
    be = 512
    grid = (E_PAD // be,)
    return pl.pallas_call(
        _combine_body_split,
        grid=grid,
        in_specs=[
            pl.BlockSpec((EDGE_ORDER, be, 128), lambda i: (0, i, 0)),
            pl.BlockSpec((EDGE_ORDER, be), lambda i: (0, i)),
            pl.BlockSpec((64, 64), lambda i: (0, 0)),
            pl.BlockSpec((1, 64), lambda i: (0, 0)),
        ],
        out_specs=pl.BlockSpec((EDGE_ORDER, be, 128), lambda i: (0, i, 0)),
        out_shape=jax.ShapeDtypeStruct((EDGE_ORDER, E_PAD, 128), jnp.float32),
    )(g, en_m, wq_pad, bq_pad)


def _final_body(s_ref, dA_ref, dB_ref, o_ref):
    deg = jnp.maximum(dA_ref[...] + dB_ref[...], 1.0)
    o_ref[...] = jnp.maximum(s_ref[...] / deg, 0.0)


def _finalize(node_sum, dA, dB):
    bn = 1000
    grid = (N_NODES // bn,)
    return pl.pallas_call(
        _final_body,
        grid=grid,
        in_specs=[
            pl.BlockSpec((bn, 64), lambda i: (i, 0)),
            pl.BlockSpec((bn, 1), lambda i: (i, 0)),
            pl.BlockSpec((bn, 1), lambda i: (i, 0)),
        ],
        out_specs=pl.BlockSpec((bn, 64), lambda i: (i, 0)),
        out_shape=jax.ShapeDtypeStruct((N_NODES, 64), jnp.float32),
    )(node_sum, dA, dB)


def kernel(embedding, edge_nodes, Wp, bp, Wq, bq, W1, b1, W2, b2):
    f32 = jnp.float32
    en = edge_nodes.astype(jnp.int32)
    en_pad = jnp.concatenate(
        [en, jnp.full((E_PAD - N_EDGES, EDGE_ORDER), N_NODES, jnp.int32)], axis=0)
    idx_flat = en_pad.T.reshape(-1)

    idx_q = idx_flat // 4
    pair_par = (idx_flat // 2) % 2
    idxE = jnp.where(pair_par == 0, idx_q, 12500)
    idxO = jnp.where(pair_par == 1, idx_q, 12500)
    emb_pad = jnp.zeros((N_PAD, FEATDIM), f32).at[:N_NODES].set(embedding)
    zeros_vec = jnp.zeros((N_PAD,), f32)
    zeros_mat = jnp.zeros((ZROWS, 128), f32)
    ones_vec = jnp.ones((M_PER_W,), f32)
    wp_pad = jnp.zeros((FEATDIM, 64), f32).at[:, :RANK].set(Wp)
    bp_pad = jnp.zeros((1, 64), f32).at[0, :RANK].set(bp)
    wq_pad = jnp.zeros((64, 64), f32).at[:RANK, :].set(Wq)
    bq_pad = bq.reshape(1, OUTDIM).astype(f32)
    b1r = b1.reshape(1, HIDDEN).astype(f32)
    b2r = b2.reshape(1, OUTDIM).astype(f32)
    eye64 = jnp.eye(64, dtype=f32)
    embL = jnp.zeros((64, 128), f32).at[:, :64].set(eye64)
    embR = jnp.zeros((64, 128), f32).at[:, 64:].set(eye64)

    deg_k, gather_k, scatter_k = _sc_kernels()
    degA, degB = deg_k(idx_flat, ones_vec, zeros_vec)
    dA = degA.reshape(N_PAD, 1)
    dB = degB.reshape(N_PAD, 1)

    c_table = _build_tables(emb_pad, dA, dB, wp_pad, bp_pad, W1, b1r, W2, b2r,
                            embL, embR)

    g = gather_k(idx_flat, c_table)

    cv = _combine(g.reshape(EDGE_ORDER, E_PAD, 128),
                  en_pad.T.reshape(EDGE_ORDER, E_PAD), wq_pad, bq_pad)

    sumE, sumO = scatter_k(idxE, idxO, cv.reshape(M_PAD, 128), zeros_mat)

    node_sum = jnp.concatenate(
        [sumE.reshape(N_PAD4, 2, OUTDIM), sumO.reshape(N_PAD4, 2, OUTDIM)],
        axis=1).reshape(N_PAD, OUTDIM)

    return _finalize(node_sum, dA, dB)
